# baseline (speedup 1.0000x reference)
import functools

import jax
import jax.numpy as jnp
from jax import lax
from jax.experimental import pallas as pl
from jax.experimental.pallas import tpu as pltpu
from jax.experimental.pallas import tpu_sc as plsc

B = 16384
D = 64
PD = 128
NC = 2
NS = 16
L = 16
NW = NC * NS
BPW = B // NW
CH = 128
NCH = BPW // CH
GPC = CH // L
CE = CH * D


def _body(uid_h, pid_h, nid_h, ut_h, it_h, pos_h, neg_h,
          idx_u, idx_p, idx_n, ixw_u, ixw_p, ixw_n,
          u_e, p_e, n_e, pos_v, neg_v, su, sp, sn):
    cid = lax.axis_index("c")
    sid = lax.axis_index("s")
    wid = sid * NC + cid
    base = wid * BPW

    pltpu.sync_copy(uid_h.at[pl.ds(base, BPW)], idx_u)
    pltpu.sync_copy(pid_h.at[pl.ds(base, BPW)], idx_p)
    pltpu.sync_copy(nid_h.at[pl.ds(base, BPW)], idx_n)

    lane = lax.iota(jnp.int32, L)
    dks = [lane + k * L for k in range(D // L)]

    ut_flat = ut_h.at[0, :]
    it_flat = it_h.at[0, :]

    for j in range(NCH):

        def build(g, carry, j=j):
            boff = pl.multiple_of(j * CH + g * L, L)
            vu = idx_u[pl.ds(boff, L)] * PD
            vp = idx_p[pl.ds(boff, L)] * PD
            vn = idx_n[pl.ds(boff, L)] * PD
            goff = pl.multiple_of(g * L * D, L)
            for l in range(L):
                ro = goff + l * D
                for k in range(D // L):
                    ixw_u[pl.ds(ro + k * L, L)] = vu[l] + dks[k]
                    ixw_p[pl.ds(ro + k * L, L)] = vp[l] + dks[k]
                    ixw_n[pl.ds(ro + k * L, L)] = vn[l] + dks[k]
            return carry

        lax.fori_loop(0, GPC, build, 0)

        cu = pltpu.async_copy(ut_flat.at[ixw_u], u_e, su)
        cp = pltpu.async_copy(it_flat.at[ixw_p], p_e, sp)
        cn = pltpu.async_copy(it_flat.at[ixw_n], n_e, sn)
        cu.wait()
        cp.wait()
        cn.wait()

        def group(g, carry, j=j):
            gbase = g * (L * D)
            lvec = lane * D
            accp = jnp.zeros((L,), jnp.float32)
            accn = jnp.zeros((L,), jnp.float32)
            for d in range(D):
                ivec = lvec + (gbase + d)
                u = plsc.load_gather(u_e, [ivec])
                p = plsc.load_gather(p_e, [ivec])
                n = plsc.load_gather(n_e, [ivec])
                accp = accp + u * p
                accn = accn + u * n
            off = pl.multiple_of(j * CH + g * L, L)
            pos_v[pl.ds(off, L)] = accp
            neg_v[pl.ds(off, L)] = accn
            return carry

        lax.fori_loop(0, GPC, group, 0)

    pltpu.sync_copy(pos_v, pos_h.at[pl.ds(base, BPW)])
    pltpu.sync_copy(neg_v, neg_h.at[pl.ds(base, BPW)])


def kernel(user_ids, pos_item_ids, neg_item_ids, user_table, item_table):
    mesh = plsc.VectorSubcoreMesh(core_axis_name="c", subcore_axis_name="s")
    f = functools.partial(
        pl.kernel,
        mesh=mesh,
        compiler_params=pltpu.CompilerParams(needs_layout_passes=False),
        out_type=(
            jax.ShapeDtypeStruct((B,), jnp.float32),
            jax.ShapeDtypeStruct((B,), jnp.float32),
        ),
        scratch_types=[
            pltpu.VMEM((BPW,), jnp.int32),
            pltpu.VMEM((BPW,), jnp.int32),
            pltpu.VMEM((BPW,), jnp.int32),
            pltpu.VMEM((CE,), jnp.int32),
            pltpu.VMEM((CE,), jnp.int32),
            pltpu.VMEM((CE,), jnp.int32),
            pltpu.VMEM((CE,), jnp.float32),
            pltpu.VMEM((CE,), jnp.float32),
            pltpu.VMEM((CE,), jnp.float32),
            pltpu.VMEM((BPW,), jnp.float32),
            pltpu.VMEM((BPW,), jnp.float32),
            pltpu.SemaphoreType.DMA,
            pltpu.SemaphoreType.DMA,
            pltpu.SemaphoreType.DMA,
        ],
    )(_body)
    return f(
        user_ids.astype(jnp.int32),
        pos_item_ids.astype(jnp.int32),
        neg_item_ids.astype(jnp.int32),
        user_table,
        item_table,
    )


# speedup vs baseline: 1.1776x; 1.1776x over previous
import functools

import jax
import jax.numpy as jnp
from jax import lax
from jax.experimental import pallas as pl
from jax.experimental.pallas import tpu as pltpu
from jax.experimental.pallas import tpu_sc as plsc

B = 16384
D = 64
NC = 2
NS = 16
L = 16
NW = NC * NS
BPW = B // NW
CH = 128
NCH = BPW // CH
GPC = CH // L


def _body(uid_h, pid_h, nid_h, ut_h, it_h, pos_h, neg_h,
          idx_u, idx_p, idx_n, ixw_u, ixw_p, ixw_n,
          u_rows, p_rows, n_rows, pos_v, neg_v, su, sp, sn):
    cid = lax.axis_index("c")
    sid = lax.axis_index("s")
    wid = sid * NC + cid
    base = wid * BPW

    pltpu.sync_copy(uid_h.at[pl.ds(base, BPW)], idx_u)
    pltpu.sync_copy(pid_h.at[pl.ds(base, BPW)], idx_p)
    pltpu.sync_copy(nid_h.at[pl.ds(base, BPW)], idx_n)

    lane = lax.iota(jnp.int32, L)

    ut_flat = ut_h.at[pl.ds(0, 1), :]
    it_flat = it_h.at[pl.ds(0, 1), :]

    for j in range(NCH):

        def build(g, carry, j=j):
            boff = pl.multiple_of(j * CH + g * L, L)
            pos = lane * 2 + g * (2 * L)
            vu = idx_u[pl.ds(boff, L)] * 2
            vp = idx_p[pl.ds(boff, L)] * 2
            vn = idx_n[pl.ds(boff, L)] * 2
            plsc.store_scatter(ixw_u, [pos], vu)
            plsc.store_scatter(ixw_u, [pos + 1], vu + 1)
            plsc.store_scatter(ixw_p, [pos], vp)
            plsc.store_scatter(ixw_p, [pos + 1], vp + 1)
            plsc.store_scatter(ixw_n, [pos], vn)
            plsc.store_scatter(ixw_n, [pos + 1], vn + 1)
            return carry

        lax.fori_loop(0, GPC, build, 0)

        cu = pltpu.async_copy(ut_flat.at[ixw_u], u_rows, su)
        cp = pltpu.async_copy(it_flat.at[ixw_p], p_rows, sp)
        cn = pltpu.async_copy(it_flat.at[ixw_n], n_rows, sn)
        cu.wait()
        cp.wait()
        cn.wait()

        def group(g, carry, j=j):
            rvec = lane + g * L
            accp = jnp.zeros((L,), jnp.float32)
            accn = jnp.zeros((L,), jnp.float32)
            for d in range(D):
                cvec = jnp.full((L,), d, jnp.int32)
                u = plsc.load_gather(u_rows, [rvec, cvec])
                p = plsc.load_gather(p_rows, [rvec, cvec])
                n = plsc.load_gather(n_rows, [rvec, cvec])
                accp = accp + u * p
                accn = accn + u * n
            off = pl.multiple_of(j * CH + g * L, L)
            pos_v[pl.ds(off, L)] = accp
            neg_v[pl.ds(off, L)] = accn
            return carry

        lax.fori_loop(0, GPC, group, 0)

    pltpu.sync_copy(pos_v, pos_h.at[pl.ds(base, BPW)])
    pltpu.sync_copy(neg_v, neg_h.at[pl.ds(base, BPW)])


def kernel(user_ids, pos_item_ids, neg_item_ids, user_table, item_table):
    mesh = plsc.VectorSubcoreMesh(core_axis_name="c", subcore_axis_name="s")
    f = functools.partial(
        pl.kernel,
        mesh=mesh,
        compiler_params=pltpu.CompilerParams(needs_layout_passes=False),
        out_type=(
            jax.ShapeDtypeStruct((B,), jnp.float32),
            jax.ShapeDtypeStruct((B,), jnp.float32),
        ),
        scratch_types=[
            pltpu.VMEM((BPW,), jnp.int32),
            pltpu.VMEM((BPW,), jnp.int32),
            pltpu.VMEM((BPW,), jnp.int32),
            pltpu.VMEM((2 * CH,), jnp.int32),
            pltpu.VMEM((2 * CH,), jnp.int32),
            pltpu.VMEM((2 * CH,), jnp.int32),
            pltpu.VMEM((2 * CH, D), jnp.float32),
            pltpu.VMEM((2 * CH, D), jnp.float32),
            pltpu.VMEM((2 * CH, D), jnp.float32),
            pltpu.VMEM((BPW,), jnp.float32),
            pltpu.VMEM((BPW,), jnp.float32),
            pltpu.SemaphoreType.DMA,
            pltpu.SemaphoreType.DMA,
            pltpu.SemaphoreType.DMA,
        ],
    )(_body)
    return f(
        user_ids.astype(jnp.int32),
        pos_item_ids.astype(jnp.int32),
        neg_item_ids.astype(jnp.int32),
        user_table,
        item_table,
    )


# speedup vs baseline: 1.1795x; 1.0016x over previous
import functools

import jax
import jax.numpy as jnp
from jax import lax
from jax.experimental import pallas as pl
from jax.experimental.pallas import tpu as pltpu
from jax.experimental.pallas import tpu_sc as plsc

B = 16384
D = 64
NC = 2
NS = 16
L = 16
NW = NC * NS
BPW = B // NW
RND = 256
NRND = BPW // RND
GPR = RND // L


def _body(uid_h, pid_h, nid_h, ut_h, it_h, pos_h, neg_h,
          idx_u, idx_p, idx_n, ixw_u, ixw_p, ixw_n,
          u_rows, p_rows, n_rows, pos_v, neg_v, su, sp, sn):
    cid = lax.axis_index("c")
    sid = lax.axis_index("s")
    wid = sid * NC + cid
    base = wid * BPW

    pltpu.sync_copy(uid_h.at[pl.ds(base, BPW)], idx_u)
    pltpu.sync_copy(pid_h.at[pl.ds(base, BPW)], idx_p)
    pltpu.sync_copy(nid_h.at[pl.ds(base, BPW)], idx_n)

    lane = lax.iota(jnp.int32, L)

    ut_flat = ut_h.at[pl.ds(0, 1), :]
    it_flat = it_h.at[pl.ds(0, 1), :]

    for r in range(NRND):

        def build(g, carry, r=r):
            boff = pl.multiple_of(r * RND + g * L, L)
            goff = pl.multiple_of(g * L, L)
            ixw_u[pl.ds(goff, L)] = idx_u[pl.ds(boff, L)] * 2
            ixw_p[pl.ds(goff, L)] = idx_p[pl.ds(boff, L)] * 2
            ixw_n[pl.ds(goff, L)] = idx_n[pl.ds(boff, L)] * 2
            return carry

        lax.fori_loop(0, GPR, build, 0)

        cu = pltpu.async_copy(ut_flat.at[ixw_u], u_rows, su)
        cp = pltpu.async_copy(it_flat.at[ixw_p], p_rows, sp)
        cn = pltpu.async_copy(it_flat.at[ixw_n], n_rows, sn)
        cu.wait()
        cp.wait()
        cn.wait()

        def group(g, carry, r=r):
            iv = lane + g * L
            rvec = jax.lax.shift_right_logical(iv, 1)
            cbase = (iv & 1) * D
            accp = jnp.zeros((L,), jnp.float32)
            accn = jnp.zeros((L,), jnp.float32)
            for d in range(D):
                cvec = cbase + d
                u = plsc.load_gather(u_rows, [rvec, cvec])
                p = plsc.load_gather(p_rows, [rvec, cvec])
                n = plsc.load_gather(n_rows, [rvec, cvec])
                accp = accp + u * p
                accn = accn + u * n
            off = pl.multiple_of(r * RND + g * L, L)
            pos_v[pl.ds(off, L)] = accp
            neg_v[pl.ds(off, L)] = accn
            return carry

        lax.fori_loop(0, GPR, group, 0)

    pltpu.sync_copy(pos_v, pos_h.at[pl.ds(base, BPW)])
    pltpu.sync_copy(neg_v, neg_h.at[pl.ds(base, BPW)])


def kernel(user_ids, pos_item_ids, neg_item_ids, user_table, item_table):
    mesh = plsc.VectorSubcoreMesh(core_axis_name="c", subcore_axis_name="s")
    f = functools.partial(
        pl.kernel,
        mesh=mesh,
        compiler_params=pltpu.CompilerParams(needs_layout_passes=False),
        out_type=(
            jax.ShapeDtypeStruct((B,), jnp.float32),
            jax.ShapeDtypeStruct((B,), jnp.float32),
        ),
        scratch_types=[
            pltpu.VMEM((BPW,), jnp.int32),
            pltpu.VMEM((BPW,), jnp.int32),
            pltpu.VMEM((BPW,), jnp.int32),
            pltpu.VMEM((RND,), jnp.int32),
            pltpu.VMEM((RND,), jnp.int32),
            pltpu.VMEM((RND,), jnp.int32),
            pltpu.VMEM((RND, D), jnp.float32),
            pltpu.VMEM((RND, D), jnp.float32),
            pltpu.VMEM((RND, D), jnp.float32),
            pltpu.VMEM((BPW,), jnp.float32),
            pltpu.VMEM((BPW,), jnp.float32),
            pltpu.SemaphoreType.DMA,
            pltpu.SemaphoreType.DMA,
            pltpu.SemaphoreType.DMA,
        ],
    )(_body)
    return f(
        user_ids.astype(jnp.int32),
        pos_item_ids.astype(jnp.int32),
        neg_item_ids.astype(jnp.int32),
        user_table,
        item_table,
    )
